# baseline (speedup 1.0000x reference)
import jax
import jax.numpy as jnp
from jax import lax
from jax.experimental import pallas as pl
from jax.experimental.pallas import tpu as pltpu
from jax.experimental.pallas import tpu_sc as plsc

N_NODES = 10000
N_EDGES = 320000
D = 128
N_HEADS = 4
HEAD_DIM = 32
EXT = 144
ADW = 16

NW = 32
E_PER_W = N_EDGES // NW
CHUNK = 64
N_CHUNKS = 158
E_PAD_W = N_CHUNKS * CHUNK
NBODY = N_CHUNKS // 2
ROWS_PER_TILE = 624
ROWS_TAIL = N_NODES - 16 * ROWS_PER_TILE


def _tc1_body(x_ref, wcat_ref, aext_ref, adstw_ref, hext_ref, adst_ref):
    h = lax.dot(x_ref[...], wcat_ref[...], preferred_element_type=jnp.float32)
    hext_ref[:, 0:D] = h
    hext_ref[:, D:EXT] = lax.dot(h, aext_ref[...],
                                 preferred_element_type=jnp.float32)
    adst_ref[...] = lax.dot(h, adstw_ref[...],
                            preferred_element_type=jnp.float32)


def _tc1(x, w_cat, a_ext, a_dstw):
    blk = 1000
    grid = (N_NODES // blk,)
    return pl.pallas_call(
        _tc1_body,
        grid=grid,
        in_specs=[
            pl.BlockSpec((blk, D), lambda i: (i, 0)),
            pl.BlockSpec((D, D), lambda i: (0, 0)),
            pl.BlockSpec((D, EXT - D), lambda i: (0, 0)),
            pl.BlockSpec((D, ADW), lambda i: (0, 0)),
        ],
        out_specs=[
            pl.BlockSpec((blk, EXT), lambda i: (i, 0)),
            pl.BlockSpec((blk, ADW), lambda i: (i, 0)),
        ],
        out_shape=[
            jax.ShapeDtypeStruct((N_NODES, EXT), jnp.float32),
            jax.ShapeDtypeStruct((N_NODES, ADW), jnp.float32),
        ],
    )(x, w_cat, a_ext, a_dstw)


def _sc_body(edges_hbm, hext_hbm, adst_hbm, out_hbm,
             ebuf0, ebuf1, a0, a1, ad0, ad1, b0, b1, sbuf0, sbuf1,
             acc_sh, gsem0, gsem1, esem0, esem1, ssem0, ssem1):
    c = lax.axis_index("c")
    s = lax.axis_index("s")
    w = s * 2 + c

    lane = lax.iota(jnp.int32, 16)
    zero16 = jnp.zeros((16,), jnp.float32)
    for b in (b0, b1):
        for r in range(CHUNK):
            for k in range(EXT // 16):
                b[r, pl.ds(k * 16, 16)] = zero16

    base_row = s * ROWS_PER_TILE
    for k in range(ROWS_PER_TILE // CHUNK):
        pltpu.sync_copy(b0.at[pl.ds(0, CHUNK)],
                        acc_sh.at[pl.ds(base_row + k * CHUNK, CHUNK)])
    rem = ROWS_PER_TILE % CHUNK
    pltpu.sync_copy(
        b0.at[pl.ds(0, rem)],
        acc_sh.at[pl.ds(base_row + ROWS_PER_TILE - rem, rem)])

    @pl.when(s == 15)
    def _zero_tail():
        pltpu.sync_copy(b0.at[pl.ds(0, ROWS_TAIL)],
                        acc_sh.at[pl.ds(16 * ROWS_PER_TILE, ROWS_TAIL)])

    plsc.subcore_barrier()

    def do_chunk(a, ad, b, chunk_id):
        base = chunk_id * CHUNK
        for q in range(CHUNK // 16):
            rows = lane + (q * 16)
            valid = (base + q * 16 + lane) < E_PER_W
            ex_regs = []
            for hd in range(N_HEADS):
                col = jnp.full((16,), D + hd, dtype=jnp.int32)
                a_s = plsc.load_gather(a, [rows, col])
                a_d = plsc.load_gather(ad,
                                       [rows, jnp.full((16,), hd, jnp.int32)])
                e = a_s + a_d
                e = jnp.where(e > 0, e, 0.2 * e)
                ex = jnp.where(valid, jnp.exp(e), 0.0)
                ex_regs.append(ex)
                plsc.store_scatter(b, [rows, col], ex)
            for hd in range(N_HEADS):
                ex = ex_regs[hd]

                @plsc.parallel_loop(hd * HEAD_DIM, (hd + 1) * HEAD_DIM,
                                    unroll=16)
                def _scale_col(ci):
                    colv = jnp.broadcast_to(ci, (16,)).astype(jnp.int32)
                    hv = plsc.load_gather(a, [rows, colv])
                    plsc.store_scatter(b, [rows, colv], hv * ex)

    def save_dst(ebuf, sbuf):
        for q in range(CHUNK // 16):
            sbuf[pl.ds(q * 16, 16)] = ebuf[1, pl.ds(q * 16, 16)]

    def start_gathers(ebuf, a, ad, sem):
        pltpu.async_copy(hext_hbm.at[ebuf.at[0]], a, sem)
        pltpu.async_copy(adst_hbm.at[ebuf.at[1]], ad, sem)

    def wait_gathers(ebuf, a, ad, sem):
        pltpu.make_async_copy(hext_hbm.at[ebuf.at[0]], a, sem).wait()
        pltpu.make_async_copy(adst_hbm.at[ebuf.at[1]], ad, sem).wait()

    pltpu.sync_copy(edges_hbm.at[w, 0], ebuf0)
    pltpu.sync_copy(edges_hbm.at[w, 1], ebuf1)
    start_gathers(ebuf0, a0, ad0, gsem0)
    start_gathers(ebuf1, a1, ad1, gsem1)

    def body(t, carry):
        last = NBODY - 1
        wait_gathers(ebuf0, a0, ad0, gsem0)

        @pl.when(t > 0)
        def _drain_b0():
            pltpu.make_async_copy(b0, acc_sh.at[sbuf0], ssem0).wait()

        save_dst(ebuf0, sbuf0)

        @pl.when(t < last)
        def _stage_e0():
            pltpu.async_copy(edges_hbm.at[w, 2 * t + 2], ebuf0, esem0)

        do_chunk(a0, ad0, b0, 2 * t)
        pltpu.async_copy(b0, acc_sh.at[sbuf0], ssem0, add=True)

        @pl.when(t < last)
        def _gather_next0():
            pltpu.make_async_copy(edges_hbm.at[w, 2 * t + 2],
                                  ebuf0, esem0).wait()
            start_gathers(ebuf0, a0, ad0, gsem0)

        wait_gathers(ebuf1, a1, ad1, gsem1)

        @pl.when(t > 0)
        def _drain_b1():
            pltpu.make_async_copy(b1, acc_sh.at[sbuf1], ssem1).wait()

        save_dst(ebuf1, sbuf1)

        @pl.when(t < last)
        def _stage_e1():
            pltpu.async_copy(edges_hbm.at[w, 2 * t + 3], ebuf1, esem1)

        do_chunk(a1, ad1, b1, 2 * t + 1)
        pltpu.async_copy(b1, acc_sh.at[sbuf1], ssem1, add=True)

        @pl.when(t < last)
        def _gather_next1():
            pltpu.make_async_copy(edges_hbm.at[w, 2 * t + 3],
                                  ebuf1, esem1).wait()
            start_gathers(ebuf1, a1, ad1, gsem1)

        return carry

    lax.fori_loop(0, NBODY, body, 0)
    pltpu.make_async_copy(b0, acc_sh.at[sbuf0], ssem0).wait()
    pltpu.make_async_copy(b1, acc_sh.at[sbuf1], ssem1).wait()
    plsc.subcore_barrier()
    pltpu.sync_copy(acc_sh.at[pl.ds(s * ROWS_PER_TILE, ROWS_PER_TILE)],
                    out_hbm.at[c].at[pl.ds(s * ROWS_PER_TILE, ROWS_PER_TILE)])

    @pl.when(s == 15)
    def _out_tail():
        pltpu.sync_copy(acc_sh.at[pl.ds(16 * ROWS_PER_TILE, ROWS_TAIL)],
                        out_hbm.at[c].at[pl.ds(16 * ROWS_PER_TILE, ROWS_TAIL)])


def _sc_edge_phase(edges_p, h_ext, adst):
    mesh = plsc.VectorSubcoreMesh(core_axis_name="c", subcore_axis_name="s")
    f = pl.kernel(
        _sc_body,
        out_type=jax.ShapeDtypeStruct((2, N_NODES, EXT), jnp.float32),
        mesh=mesh,
        compiler_params=pltpu.CompilerParams(use_tc_tiling_on_sc=False,
                                             needs_layout_passes=False),
        scratch_types=[
            pltpu.VMEM((2, CHUNK), jnp.int32),
            pltpu.VMEM((2, CHUNK), jnp.int32),
            pltpu.VMEM((CHUNK, EXT), jnp.float32),
            pltpu.VMEM((CHUNK, EXT), jnp.float32),
            pltpu.VMEM((CHUNK, ADW), jnp.float32),
            pltpu.VMEM((CHUNK, ADW), jnp.float32),
            pltpu.VMEM((CHUNK, EXT), jnp.float32),
            pltpu.VMEM((CHUNK, EXT), jnp.float32),
            pltpu.VMEM((CHUNK,), jnp.int32),
            pltpu.VMEM((CHUNK,), jnp.int32),
            pltpu.VMEM_SHARED((N_NODES, EXT), jnp.float32),
            pltpu.SemaphoreType.DMA,
            pltpu.SemaphoreType.DMA,
            pltpu.SemaphoreType.DMA,
            pltpu.SemaphoreType.DMA,
            pltpu.SemaphoreType.DMA,
            pltpu.SemaphoreType.DMA,
        ],
    )
    return f(edges_p, h_ext, adst)


def _tc2_body(parts_ref, x_ref, wo_ref, bo_ref, gamma_ref, beta_ref,
              bcat_ref, p_ref, out_ref):
    a = parts_ref[0] + parts_ref[1]
    s4 = a[:, D:D + N_HEADS]
    rep = lax.dot(s4, p_ref[...], preferred_element_type=jnp.float32)
    mh = a[:, 0:D] / (rep + 1e-16) + bcat_ref[...]
    y = (lax.dot(mh, wo_ref[...], preferred_element_type=jnp.float32)
         + bo_ref[...] + x_ref[...])
    mu = jnp.mean(y, axis=1, keepdims=True)
    var = jnp.mean((y - mu) ** 2, axis=1, keepdims=True)
    out_ref[...] = ((y - mu) * lax.rsqrt(var + 1e-5) * gamma_ref[...]
                    + beta_ref[...])


def _tc2(parts, x, wo, bo2, gamma2, beta2, bcat2, p):
    blk = 1000
    grid = (N_NODES // blk,)
    return pl.pallas_call(
        _tc2_body,
        grid=grid,
        in_specs=[
            pl.BlockSpec((2, blk, EXT), lambda i: (0, i, 0)),
            pl.BlockSpec((blk, D), lambda i: (i, 0)),
            pl.BlockSpec((D, D), lambda i: (0, 0)),
            pl.BlockSpec((1, D), lambda i: (0, 0)),
            pl.BlockSpec((1, D), lambda i: (0, 0)),
            pl.BlockSpec((1, D), lambda i: (0, 0)),
            pl.BlockSpec((1, D), lambda i: (0, 0)),
            pl.BlockSpec((N_HEADS, D), lambda i: (0, 0)),
        ],
        out_specs=pl.BlockSpec((blk, D), lambda i: (i, 0)),
        out_shape=jax.ShapeDtypeStruct((N_NODES, D), jnp.float32),
    )(parts, x, wo, bo2, gamma2, beta2, bcat2, p)


@jax.jit
def kernel(x, edge_index, W, a_src, a_dst, b_gat, Wo, bo, gamma, beta):
    f32 = jnp.float32
    w_cat = jnp.transpose(W, (1, 0, 2)).reshape(D, D).astype(f32)
    eye_h = jnp.eye(N_HEADS, dtype=f32)
    a_srcw = (a_src[:, :, None] * eye_h[:, None, :]).reshape(D, N_HEADS)
    a_ext = jnp.pad(a_srcw, ((0, 0), (0, EXT - D - N_HEADS)))
    a_dstw4 = (a_dst[:, :, None] * eye_h[:, None, :]).reshape(D, N_HEADS)
    a_dstw = jnp.pad(a_dstw4, ((0, 0), (0, ADW - N_HEADS)))
    p = jnp.repeat(eye_h, HEAD_DIM, axis=1)
    bcat2 = b_gat.reshape(1, D).astype(f32)
    bo2 = bo.reshape(1, D).astype(f32)
    gamma2 = gamma.reshape(1, D).astype(f32)
    beta2 = beta.reshape(1, D).astype(f32)

    src = edge_index[0].astype(jnp.int32).reshape(NW, E_PER_W)
    dst = edge_index[1].astype(jnp.int32).reshape(NW, E_PER_W)
    npad = E_PAD_W - E_PER_W
    padv = jnp.broadcast_to((jnp.arange(npad, dtype=jnp.int32) * 37) % N_NODES,
                            (NW, npad))
    src_p = jnp.concatenate([src, padv], axis=1).reshape(NW, N_CHUNKS, CHUNK)
    dst_p = jnp.concatenate([dst, padv], axis=1).reshape(NW, N_CHUNKS, CHUNK)
    edges_p = jnp.stack([src_p, dst_p], axis=2)

    h_ext, adst = _tc1(x.astype(f32), w_cat, a_ext, a_dstw)
    parts = _sc_edge_phase(edges_p, h_ext, adst)
    return _tc2(parts, x.astype(f32), Wo.astype(f32), bo2, gamma2, beta2,
                bcat2, p)


# speedup vs baseline: 1.1509x; 1.1509x over previous
import jax
import jax.numpy as jnp
from jax import lax
from jax.experimental import pallas as pl
from jax.experimental.pallas import tpu as pltpu
from jax.experimental.pallas import tpu_sc as plsc

N_NODES = 10000
N_EDGES = 320000
D = 128
N_HEADS = 4
HEAD_DIM = 32
EXT = 144
ADW = 16

NW = 32
E_PER_W = N_EDGES // NW
CHUNK = 64
N_CHUNKS = 158
E_PAD_W = N_CHUNKS * CHUNK
NBODY = N_CHUNKS // 2
ROWS_PER_TILE = 624
ROWS_TAIL = N_NODES - 16 * ROWS_PER_TILE


def _tc1_body(x_ref, wcat_ref, aext_ref, adstw_ref, hext_ref, adst_ref):
    h = lax.dot(x_ref[...], wcat_ref[...], preferred_element_type=jnp.float32)
    hext_ref[:, 0:D] = h
    hext_ref[:, D:EXT] = lax.dot(h, aext_ref[...],
                                 preferred_element_type=jnp.float32)
    adst_ref[...] = lax.dot(h, adstw_ref[...],
                            preferred_element_type=jnp.float32)


def _tc1(x, w_cat, a_ext, a_dstw):
    blk = 1000
    grid = (N_NODES // blk,)
    return pl.pallas_call(
        _tc1_body,
        grid=grid,
        in_specs=[
            pl.BlockSpec((blk, D), lambda i: (i, 0)),
            pl.BlockSpec((D, D), lambda i: (0, 0)),
            pl.BlockSpec((D, EXT - D), lambda i: (0, 0)),
            pl.BlockSpec((D, ADW), lambda i: (0, 0)),
        ],
        out_specs=[
            pl.BlockSpec((blk, EXT), lambda i: (i, 0)),
            pl.BlockSpec((blk, ADW), lambda i: (i, 0)),
        ],
        out_shape=[
            jax.ShapeDtypeStruct((N_NODES, EXT), jnp.float32),
            jax.ShapeDtypeStruct((N_NODES, ADW), jnp.float32),
        ],
    )(x, w_cat, a_ext, a_dstw)


def _sc_body(edges_hbm, hext_hbm, adst_hbm, out_hbm,
             ebuf0, ebuf1, a0, a1, ad0, ad1, b0, b1, sbuf0, sbuf1,
             acc_sh, gsem0, gsem1, esem0, esem1, ssem0, ssem1):
    c = lax.axis_index("c")
    s = lax.axis_index("s")
    w = s * 2 + c

    lane = lax.iota(jnp.int32, 16)
    zero16 = jnp.zeros((16,), jnp.float32)
    for b in (b0, b1):
        for r in range(CHUNK):
            for k in range(EXT // 16):
                b[r, pl.ds(k * 16, 16)] = zero16

    base_row = s * ROWS_PER_TILE
    for k in range(ROWS_PER_TILE // CHUNK):
        pltpu.sync_copy(b0.at[pl.ds(0, CHUNK)],
                        acc_sh.at[pl.ds(base_row + k * CHUNK, CHUNK)])
    rem = ROWS_PER_TILE % CHUNK
    pltpu.sync_copy(
        b0.at[pl.ds(0, rem)],
        acc_sh.at[pl.ds(base_row + ROWS_PER_TILE - rem, rem)])

    @pl.when(s == 15)
    def _zero_tail():
        pltpu.sync_copy(b0.at[pl.ds(0, ROWS_TAIL)],
                        acc_sh.at[pl.ds(16 * ROWS_PER_TILE, ROWS_TAIL)])

    plsc.subcore_barrier()

    def do_chunk(a, ad, b, chunk_id):
        lo = chunk_id * CHUNK
        off_local = jnp.minimum(lo, E_PER_W - CHUNK)
        for q in range(CHUNK // 16):
            rows = lane + (q * 16)
            p = off_local + q * 16 + lane
            valid = (p >= lo) & (p < E_PER_W)
            ex_regs = []
            for hd in range(N_HEADS):
                col = jnp.full((16,), D + hd, dtype=jnp.int32)
                a_s = plsc.load_gather(a, [rows, col])
                a_d = plsc.load_gather(ad,
                                       [rows, jnp.full((16,), hd, jnp.int32)])
                e = a_s + a_d
                e = jnp.where(e > 0, e, 0.2 * e)
                ex = jnp.where(valid, jnp.exp(e), 0.0)
                ex_regs.append(ex)
                plsc.store_scatter(b, [rows, col], ex)
            for hd in range(N_HEADS):
                ex = ex_regs[hd]

                @plsc.parallel_loop(hd * HEAD_DIM, (hd + 1) * HEAD_DIM,
                                    unroll=8)
                def _scale_col(ci):
                    colv = jnp.broadcast_to(ci, (16,)).astype(jnp.int32)
                    hv = plsc.load_gather(a, [rows, colv])
                    plsc.store_scatter(b, [rows, colv], hv * ex)

    def save_dst(ebuf, sbuf):
        for q in range(CHUNK // 16):
            sbuf[pl.ds(q * 16, 16)] = ebuf[1, pl.ds(q * 16, 16)]

    def start_gathers(ebuf, a, ad, sem):
        pltpu.async_copy(hext_hbm.at[ebuf.at[0]], a, sem)
        pltpu.async_copy(adst_hbm.at[ebuf.at[1]], ad, sem)

    def wait_gathers(ebuf, a, ad, sem):
        pltpu.make_async_copy(hext_hbm.at[ebuf.at[0]], a, sem).wait()
        pltpu.make_async_copy(adst_hbm.at[ebuf.at[1]], ad, sem).wait()

    def idx_off(cj):
        return w * E_PER_W + jnp.minimum(cj * CHUNK, E_PER_W - CHUNK)

    def stage_idx(cj, ebuf, sem):
        off = idx_off(cj)
        pltpu.async_copy(edges_hbm.at[0, pl.ds(off, CHUNK)], ebuf.at[0], sem)
        pltpu.async_copy(edges_hbm.at[1, pl.ds(off, CHUNK)], ebuf.at[1], sem)

    def wait_idx(cj, ebuf, sem):
        off = idx_off(cj)
        pltpu.make_async_copy(edges_hbm.at[0, pl.ds(off, CHUNK)],
                              ebuf.at[0], sem).wait()
        pltpu.make_async_copy(edges_hbm.at[1, pl.ds(off, CHUNK)],
                              ebuf.at[1], sem).wait()

    stage_idx(0, ebuf0, esem0)
    wait_idx(0, ebuf0, esem0)
    stage_idx(1, ebuf1, esem1)
    wait_idx(1, ebuf1, esem1)
    start_gathers(ebuf0, a0, ad0, gsem0)
    start_gathers(ebuf1, a1, ad1, gsem1)

    def body(t, carry):
        last = NBODY - 1
        wait_gathers(ebuf0, a0, ad0, gsem0)

        @pl.when(t > 0)
        def _drain_b0():
            pltpu.make_async_copy(b0, acc_sh.at[sbuf0], ssem0).wait()

        save_dst(ebuf0, sbuf0)

        @pl.when(t < last)
        def _stage_e0():
            stage_idx(2 * t + 2, ebuf0, esem0)

        do_chunk(a0, ad0, b0, 2 * t)
        pltpu.async_copy(b0, acc_sh.at[sbuf0], ssem0, add=True)

        @pl.when(t < last)
        def _gather_next0():
            wait_idx(2 * t + 2, ebuf0, esem0)
            start_gathers(ebuf0, a0, ad0, gsem0)

        wait_gathers(ebuf1, a1, ad1, gsem1)

        @pl.when(t > 0)
        def _drain_b1():
            pltpu.make_async_copy(b1, acc_sh.at[sbuf1], ssem1).wait()

        save_dst(ebuf1, sbuf1)

        @pl.when(t < last)
        def _stage_e1():
            stage_idx(2 * t + 3, ebuf1, esem1)

        do_chunk(a1, ad1, b1, 2 * t + 1)
        pltpu.async_copy(b1, acc_sh.at[sbuf1], ssem1, add=True)

        @pl.when(t < last)
        def _gather_next1():
            wait_idx(2 * t + 3, ebuf1, esem1)
            start_gathers(ebuf1, a1, ad1, gsem1)

        return carry

    lax.fori_loop(0, NBODY, body, 0)
    pltpu.make_async_copy(b0, acc_sh.at[sbuf0], ssem0).wait()
    pltpu.make_async_copy(b1, acc_sh.at[sbuf1], ssem1).wait()
    plsc.subcore_barrier()
    pltpu.sync_copy(acc_sh.at[pl.ds(s * ROWS_PER_TILE, ROWS_PER_TILE)],
                    out_hbm.at[c].at[pl.ds(s * ROWS_PER_TILE, ROWS_PER_TILE)])

    @pl.when(s == 15)
    def _out_tail():
        pltpu.sync_copy(acc_sh.at[pl.ds(16 * ROWS_PER_TILE, ROWS_TAIL)],
                        out_hbm.at[c].at[pl.ds(16 * ROWS_PER_TILE, ROWS_TAIL)])


def _sc_edge_phase(edges_p, h_ext, adst):
    mesh = plsc.VectorSubcoreMesh(core_axis_name="c", subcore_axis_name="s")
    f = pl.kernel(
        _sc_body,
        out_type=jax.ShapeDtypeStruct((2, N_NODES, EXT), jnp.float32),
        mesh=mesh,
        compiler_params=pltpu.CompilerParams(use_tc_tiling_on_sc=False,
                                             needs_layout_passes=False),
        scratch_types=[
            pltpu.VMEM((2, CHUNK), jnp.int32),
            pltpu.VMEM((2, CHUNK), jnp.int32),
            pltpu.VMEM((CHUNK, EXT), jnp.float32),
            pltpu.VMEM((CHUNK, EXT), jnp.float32),
            pltpu.VMEM((CHUNK, ADW), jnp.float32),
            pltpu.VMEM((CHUNK, ADW), jnp.float32),
            pltpu.VMEM((CHUNK, EXT), jnp.float32),
            pltpu.VMEM((CHUNK, EXT), jnp.float32),
            pltpu.VMEM((CHUNK,), jnp.int32),
            pltpu.VMEM((CHUNK,), jnp.int32),
            pltpu.VMEM_SHARED((N_NODES, EXT), jnp.float32),
            pltpu.SemaphoreType.DMA,
            pltpu.SemaphoreType.DMA,
            pltpu.SemaphoreType.DMA,
            pltpu.SemaphoreType.DMA,
            pltpu.SemaphoreType.DMA,
            pltpu.SemaphoreType.DMA,
        ],
    )
    return f(edges_p, h_ext, adst)


def _tc2_body(parts_ref, x_ref, wo_ref, bo_ref, gamma_ref, beta_ref,
              bcat_ref, p_ref, out_ref):
    a = parts_ref[0] + parts_ref[1]
    s4 = a[:, D:D + N_HEADS]
    rep = lax.dot(s4, p_ref[...], preferred_element_type=jnp.float32)
    mh = a[:, 0:D] / (rep + 1e-16) + bcat_ref[...]
    y = (lax.dot(mh, wo_ref[...], preferred_element_type=jnp.float32)
         + bo_ref[...] + x_ref[...])
    mu = jnp.mean(y, axis=1, keepdims=True)
    var = jnp.mean((y - mu) ** 2, axis=1, keepdims=True)
    out_ref[...] = ((y - mu) * lax.rsqrt(var + 1e-5) * gamma_ref[...]
                    + beta_ref[...])


def _tc2(parts, x, wo, bo2, gamma2, beta2, bcat2, p):
    blk = 1000
    grid = (N_NODES // blk,)
    return pl.pallas_call(
        _tc2_body,
        grid=grid,
        in_specs=[
            pl.BlockSpec((2, blk, EXT), lambda i: (0, i, 0)),
            pl.BlockSpec((blk, D), lambda i: (i, 0)),
            pl.BlockSpec((D, D), lambda i: (0, 0)),
            pl.BlockSpec((1, D), lambda i: (0, 0)),
            pl.BlockSpec((1, D), lambda i: (0, 0)),
            pl.BlockSpec((1, D), lambda i: (0, 0)),
            pl.BlockSpec((1, D), lambda i: (0, 0)),
            pl.BlockSpec((N_HEADS, D), lambda i: (0, 0)),
        ],
        out_specs=pl.BlockSpec((blk, D), lambda i: (i, 0)),
        out_shape=jax.ShapeDtypeStruct((N_NODES, D), jnp.float32),
    )(parts, x, wo, bo2, gamma2, beta2, bcat2, p)


@jax.jit
def kernel(x, edge_index, W, a_src, a_dst, b_gat, Wo, bo, gamma, beta):
    f32 = jnp.float32
    w_cat = jnp.transpose(W, (1, 0, 2)).reshape(D, D).astype(f32)
    eye_h = jnp.eye(N_HEADS, dtype=f32)
    a_srcw = (a_src[:, :, None] * eye_h[:, None, :]).reshape(D, N_HEADS)
    a_ext = jnp.pad(a_srcw, ((0, 0), (0, EXT - D - N_HEADS)))
    a_dstw4 = (a_dst[:, :, None] * eye_h[:, None, :]).reshape(D, N_HEADS)
    a_dstw = jnp.pad(a_dstw4, ((0, 0), (0, ADW - N_HEADS)))
    p = jnp.repeat(eye_h, HEAD_DIM, axis=1)
    bcat2 = b_gat.reshape(1, D).astype(f32)
    bo2 = bo.reshape(1, D).astype(f32)
    gamma2 = gamma.reshape(1, D).astype(f32)
    beta2 = beta.reshape(1, D).astype(f32)

    edges32 = edge_index.astype(jnp.int32)

    h_ext, adst = _tc1(x.astype(f32), w_cat, a_ext, a_dstw)
    parts = _sc_edge_phase(edges32, h_ext, adst)
    return _tc2(parts, x.astype(f32), Wo.astype(f32), bo2, gamma2, beta2,
                bcat2, p)
